# Initial kernel scaffold; baseline (speedup 1.0000x reference)
import functools

import jax
import jax.numpy as jnp
from jax import lax
from jax.experimental import pallas as pl
from jax.experimental.pallas import tpu as pltpu
from jax.experimental.pallas import tpu_sc as plsc

N = 10000
E = 320000
D = 128
C = 128
SLOPE = 0.2

NP = 10240
EP = 327680
ACC_COLS = 144

NW = 32
CHUNK = EP // NW
BLK = 256
NBLK = CHUNK // BLK

F32 = jnp.float32
I32 = jnp.int32


def _mm_body(x_ref, w_ref, a_ref, xo_ref, ao_ref):
    x = x_ref[...]
    xw = jnp.dot(x, w_ref[...], preferred_element_type=F32)
    xo_ref[...] = xw
    a_row = a_ref[0:1, :]
    a1d = jnp.sum(xw * a_row, axis=1)
    ao_ref[...] = a1d.reshape(ao_ref.shape)


def _tc_project(x, w, attn_vec, rows_blk):
    rows = x.shape[0]
    grid = rows // rows_blk
    a8 = jnp.broadcast_to(attn_vec.reshape(1, D), (8, D))
    return pl.pallas_call(
        _mm_body,
        grid=(grid,),
        in_specs=[
            pl.BlockSpec((rows_blk, D), lambda i: (i, 0)),
            pl.BlockSpec((D, D), lambda i: (0, 0)),
            pl.BlockSpec((8, D), lambda i: (0, 0)),
        ],
        out_specs=[
            pl.BlockSpec((rows_blk, D), lambda i: (i, 0)),
            pl.BlockSpec((rows_blk // 128, 128), lambda i: (i, 0)),
        ],
        out_shape=[
            jax.ShapeDtypeStruct((rows, D), F32),
            jax.ShapeDtypeStruct((rows // 128, 128), F32),
        ],
    )(x, w, a8)


def _sc_body(xdst_hbm, ar_hbm, al_hbm, seg_hbm, seg2d_hbm, zero_hbm,
             out_hbm, al_v, xd_v, pay_v, idx_v, segf_v, ar_v, w_v):
    c = lax.axis_index("c")
    s = lax.axis_index("s")
    wid = s * 2 + c
    rows_per_tile = NP // 16
    iota = lax.iota(I32, 16)
    zi16 = jnp.zeros((16,), I32)
    zf16 = jnp.zeros((16,), F32)

    pltpu.sync_copy(zero_hbm.at[pl.ds(s * rows_per_tile, rows_per_tile)],
                    out_hbm.at[c, pl.ds(s * rows_per_tile, rows_per_tile)])
    pltpu.sync_copy(al_hbm, al_v)

    def _zpad(r, carry):
        plsc.store_scatter(pay_v, [zi16 + r, iota + 128], zf16)
        return carry
    lax.fori_loop(0, BLK, _zpad, 0)
    plsc.subcore_barrier()

    base0 = wid * CHUNK

    def _block(blk, carry):
        base = base0 + blk * BLK
        pltpu.sync_copy(xdst_hbm.at[pl.ds(base, BLK)], xd_v)
        pltpu.sync_copy(ar_hbm.at[pl.ds(base, BLK)], ar_v)
        pltpu.sync_copy(seg_hbm.at[pl.ds(base, BLK)], segf_v)
        pltpu.sync_copy(seg2d_hbm.at[pl.ds(base // 128, BLK // 128)], idx_v)

        for k in range(BLK // 16):
            sl = pl.ds(k * 16, 16)
            alv = plsc.load_gather(al_v, [segf_v[sl]])
            z = alv + ar_v[sl]
            w_v[sl] = jnp.exp(jnp.maximum(z, SLOPE * z))

        def _edge(e, carry2):
            ws = w_v[e]
            vw = zf16 + ws
            rows = zi16 + e
            for k8 in range(8):
                cols = iota + (k8 * 16)
                xv = plsc.load_gather(xd_v, [rows, cols])
                plsc.store_scatter(pay_v, [rows, cols], xv * vw)
            plsc.store_scatter(pay_v, [rows, zi16 + 128], vw)
            return carry2
        lax.fori_loop(0, BLK, _edge, 0)

        for j in range(BLK // 128):
            pltpu.sync_copy(pay_v.at[pl.ds(j * 128, 128)],
                            out_hbm.at[c].at[idx_v.at[j]],
                            add=True)
        return carry

    lax.fori_loop(0, NBLK, _block, 0)


def _sc_aggregate(x_dst, ar_flat, al_flat, seg_p, seg2d):
    mesh = plsc.VectorSubcoreMesh(core_axis_name="c", subcore_axis_name="s")
    zero = jnp.zeros((NP, ACC_COLS), F32)
    kern = functools.partial(
        pl.kernel,
        out_type=jax.ShapeDtypeStruct((2, NP, ACC_COLS), F32),
        mesh=mesh,
        scratch_types=[
            pltpu.VMEM((NP,), F32),
            pltpu.VMEM((BLK, D), F32),
            pltpu.VMEM((BLK, ACC_COLS), F32),
            pltpu.VMEM((BLK // 128, 128), I32),
            pltpu.VMEM((BLK,), I32),
            pltpu.VMEM((BLK,), F32),
            pltpu.VMEM((BLK,), F32),
        ],
    )(_sc_body)
    return kern(x_dst, ar_flat, al_flat, seg_p, seg2d, zero)


def _final_body(acc_ref, xs_ref, sa_ref, da_ref, o_ref):
    a = acc_ref[0] + acc_ref[1]
    vec = a[:, :D]
    norm = a[:, D:D + 1]
    xs = xs_ref[...]
    wsum = sa_ref[0:1, :] + da_ref[0:1, :]
    s2 = jnp.sum(xs * wsum, axis=1, keepdims=True)
    self_un = jnp.exp(jnp.maximum(s2, SLOPE * s2))
    o_ref[...] = (vec + self_un * xs) / (norm + self_un)


def _tc_final(acc, x_src, sa, da):
    rows_blk = 1024
    grid = NP // rows_blk
    sa8 = jnp.broadcast_to(sa.reshape(1, D), (8, D))
    da8 = jnp.broadcast_to(da.reshape(1, D), (8, D))
    return pl.pallas_call(
        _final_body,
        grid=(grid,),
        in_specs=[
            pl.BlockSpec((2, rows_blk, ACC_COLS), lambda i: (0, i, 0)),
            pl.BlockSpec((rows_blk, D), lambda i: (i, 0)),
            pl.BlockSpec((8, D), lambda i: (0, 0)),
            pl.BlockSpec((8, D), lambda i: (0, 0)),
        ],
        out_specs=pl.BlockSpec((rows_blk, D), lambda i: (i, 0)),
        out_shape=jax.ShapeDtypeStruct((NP, D), F32),
    )(acc, x_src, sa8, da8)


def kernel(src, edge, dst, segment_ids, W, src_attn, dst_attn):
    del edge
    sa = src_attn.reshape(D)
    da = dst_attn.reshape(D)

    dst_p = jnp.concatenate([dst, jnp.zeros((EP - E, D), F32)], axis=0)
    src_p = jnp.concatenate([src, jnp.zeros((NP - N, D), F32)], axis=0)
    seg_p = jnp.concatenate(
        [segment_ids.astype(I32), jnp.full((EP - E,), N, I32)], axis=0)
    seg2d = seg_p.reshape(EP // 128, 128)

    x_dst, ar2d = _tc_project(dst_p, W, da, 4096)
    x_src, al2d = _tc_project(src_p, W, sa, 2048)

    acc = _sc_aggregate(x_dst, ar2d.reshape(EP), al2d.reshape(NP),
                        seg_p, seg2d)
    out = _tc_final(acc, x_src, sa, da)
    return out[:N]


# speedup vs baseline: 6.9283x; 6.9283x over previous
import functools

import jax
import jax.numpy as jnp
from jax import lax
from jax.experimental import pallas as pl
from jax.experimental.pallas import tpu as pltpu
from jax.experimental.pallas import tpu_sc as plsc

N = 10000
E = 320000
D = 128
C = 128
SLOPE = 0.2

NP = 10240
EP = 327680
ACC_COLS = 144

NW = 32
CHUNK = EP // NW
BLK = 128
NBLK = CHUNK // BLK

F32 = jnp.float32
I32 = jnp.int32


def _mm_body(x_ref, w_ref, a_ref, xo_ref, ao_ref):
    x = x_ref[...]
    xw = jnp.dot(x, w_ref[...], preferred_element_type=F32)
    xo_ref[...] = xw
    a_row = a_ref[0:1, :]
    a1d = jnp.sum(xw * a_row, axis=1)
    ao_ref[...] = a1d.reshape(ao_ref.shape)


def _tc_project(x, w, attn_vec, rows_blk):
    rows = x.shape[0]
    grid = rows // rows_blk
    a8 = jnp.broadcast_to(attn_vec.reshape(1, D), (8, D))
    return pl.pallas_call(
        _mm_body,
        grid=(grid,),
        in_specs=[
            pl.BlockSpec((rows_blk, D), lambda i: (i, 0)),
            pl.BlockSpec((D, D), lambda i: (0, 0)),
            pl.BlockSpec((8, D), lambda i: (0, 0)),
        ],
        out_specs=[
            pl.BlockSpec((rows_blk, D), lambda i: (i, 0)),
            pl.BlockSpec((rows_blk // 128, 128), lambda i: (i, 0)),
        ],
        out_shape=[
            jax.ShapeDtypeStruct((rows, D), F32),
            jax.ShapeDtypeStruct((rows // 128, 128), F32),
        ],
    )(x, w, a8)


def _sc_body(xdst_hbm, ar_hbm, al_hbm, seg2d_hbm, zerov_hbm, zeron_hbm,
             outv_hbm, outn_hbm, al_v, xd_v, idx_v, ar_v, w_v, wp_v,
             accv_sh, accn_sh):
    c = lax.axis_index("c")
    s = lax.axis_index("s")
    wid = s * 2 + c
    rows_per_tile = NP // 16
    rtile = pl.ds(s * rows_per_tile, rows_per_tile)
    iota = lax.iota(I32, 16)
    zf16 = jnp.zeros((16,), F32)
    lane0 = (iota == 0).astype(F32)

    pltpu.sync_copy(zerov_hbm.at[rtile], accv_sh.at[rtile])
    pltpu.sync_copy(zeron_hbm.at[rtile], accn_sh.at[rtile])
    pltpu.sync_copy(al_hbm, al_v)
    plsc.subcore_barrier()

    base0 = wid * CHUNK

    def _block(blk, carry):
        base = base0 + blk * BLK
        pltpu.sync_copy(xdst_hbm.at[pl.ds(base, BLK)], xd_v)
        pltpu.sync_copy(ar_hbm.at[pl.ds(base, BLK)], ar_v)
        pltpu.sync_copy(seg2d_hbm.at[wid * NBLK + blk], idx_v)

        for k in range(BLK // 16):
            sl = pl.ds(k * 16, 16)
            seg16 = idx_v[0, sl]
            alv = plsc.load_gather(al_v, [seg16])
            z = alv + ar_v[sl]
            w_v[sl] = jnp.exp(jnp.maximum(z, SLOPE * z))

        def _egroup(g, carry2):
            wvec = w_v[pl.ds(g * 16, 16)]
            for t in range(16):
                e = g * 16 + t
                vw = zf16 + wvec[t]
                for k8 in range(8):
                    sl = pl.ds(k8 * 16, 16)
                    xd_v[e, sl] = xd_v[e, sl] * vw
                wp_v[e, pl.ds(0, 16)] = vw * lane0
            return carry2
        lax.fori_loop(0, BLK // 16, _egroup, 0)

        pltpu.sync_copy(xd_v, accv_sh.at[idx_v.at[0]], add=True)
        pltpu.sync_copy(wp_v, accn_sh.at[idx_v.at[0]], add=True)
        return carry

    lax.fori_loop(0, NBLK, _block, 0)

    plsc.subcore_barrier()
    pltpu.sync_copy(accv_sh.at[rtile], outv_hbm.at[c, rtile])
    pltpu.sync_copy(accn_sh.at[rtile], outn_hbm.at[c, rtile])


def _sc_aggregate(x_dst, ar_flat, al_flat, seg2d):
    mesh = plsc.VectorSubcoreMesh(core_axis_name="c", subcore_axis_name="s")
    zerov = jnp.zeros((NP, D), F32)
    zeron = jnp.zeros((NP, 16), F32)
    kern = functools.partial(
        pl.kernel,
        out_type=(jax.ShapeDtypeStruct((2, NP, D), F32),
                  jax.ShapeDtypeStruct((2, NP, 16), F32)),
        mesh=mesh,
        compiler_params=pltpu.CompilerParams(use_tc_tiling_on_sc=False,
                                             needs_layout_passes=False),
        scratch_types=[
            pltpu.VMEM((NP,), F32),
            pltpu.VMEM((BLK, D), F32),
            pltpu.VMEM((1, 128), I32),
            pltpu.VMEM((BLK,), F32),
            pltpu.VMEM((BLK,), F32),
            pltpu.VMEM((BLK, 16), F32),
            pltpu.VMEM_SHARED((NP, D), F32),
            pltpu.VMEM_SHARED((NP, 16), F32),
        ],
    )(_sc_body)
    return kern(x_dst, ar_flat, al_flat, seg2d, zerov, zeron)


def _final_body(accv_ref, accn_ref, xs_ref, sa_ref, da_ref, o_ref):
    vec = accv_ref[0] + accv_ref[1]
    an = accn_ref[0] + accn_ref[1]
    norm = an[:, 0:1]
    xs = xs_ref[...]
    wsum = sa_ref[0:1, :] + da_ref[0:1, :]
    s2 = jnp.sum(xs * wsum, axis=1, keepdims=True)
    self_un = jnp.exp(jnp.maximum(s2, SLOPE * s2))
    o_ref[...] = (vec + self_un * xs) / (norm + self_un)


def _tc_final(accv, accn, x_src, sa, da):
    rows_blk = 1024
    grid = NP // rows_blk
    sa8 = jnp.broadcast_to(sa.reshape(1, D), (8, D))
    da8 = jnp.broadcast_to(da.reshape(1, D), (8, D))
    return pl.pallas_call(
        _final_body,
        grid=(grid,),
        in_specs=[
            pl.BlockSpec((2, rows_blk, D), lambda i: (0, i, 0)),
            pl.BlockSpec((2, rows_blk, 16), lambda i: (0, i, 0)),
            pl.BlockSpec((rows_blk, D), lambda i: (i, 0)),
            pl.BlockSpec((8, D), lambda i: (0, 0)),
            pl.BlockSpec((8, D), lambda i: (0, 0)),
        ],
        out_specs=pl.BlockSpec((rows_blk, D), lambda i: (i, 0)),
        out_shape=jax.ShapeDtypeStruct((NP, D), F32),
    )(accv, accn, x_src, sa8, da8)


def kernel(src, edge, dst, segment_ids, W, src_attn, dst_attn):
    del edge
    sa = src_attn.reshape(D)
    da = dst_attn.reshape(D)

    dst_p = jnp.concatenate([dst, jnp.zeros((EP - E, D), F32)], axis=0)
    src_p = jnp.concatenate([src, jnp.zeros((NP - N, D), F32)], axis=0)
    seg_p = jnp.concatenate(
        [segment_ids.astype(I32), jnp.full((EP - E,), N, I32)], axis=0)
    seg2d = seg_p.reshape(NW * NBLK, 1, 128)

    x_dst, ar2d = _tc_project(dst_p, W, da, 4096)
    x_src, al2d = _tc_project(src_p, W, sa, 2048)

    accv, accn = _sc_aggregate(x_dst, ar2d.reshape(EP), al2d.reshape(NP),
                               seg2d)
    out = _tc_final(accv, accn, x_src, sa, da)
    return out[:N]


# speedup vs baseline: 8.1966x; 1.1831x over previous
import functools

import jax
import jax.numpy as jnp
from jax import lax
from jax.experimental import pallas as pl
from jax.experimental.pallas import tpu as pltpu
from jax.experimental.pallas import tpu_sc as plsc

N = 10000
E = 320000
D = 128
C = 128
SLOPE = 0.2

NP = 10240
EP = 327680
ACC_COLS = 144

NW = 32
CHUNK = EP // NW
BLK = 128
NBLK = CHUNK // BLK

F32 = jnp.float32
I32 = jnp.int32


def _mm_body(x_ref, w_ref, a_ref, xo_ref, ao_ref):
    x = x_ref[...]
    xw = jnp.dot(x, w_ref[...], preferred_element_type=F32)
    xo_ref[...] = xw
    a_row = a_ref[0:1, :]
    a1d = jnp.sum(xw * a_row, axis=1)
    ao_ref[...] = a1d.reshape(ao_ref.shape)


def _tc_project(x, w, attn_vec, rows_blk, rows_out):
    rows = x.shape[0]
    grid = (rows + rows_blk - 1) // rows_blk
    a8 = jnp.broadcast_to(attn_vec.reshape(1, D), (8, D))
    return pl.pallas_call(
        _mm_body,
        grid=(grid,),
        in_specs=[
            pl.BlockSpec((rows_blk, D), lambda i: (i, 0)),
            pl.BlockSpec((D, D), lambda i: (0, 0)),
            pl.BlockSpec((8, D), lambda i: (0, 0)),
        ],
        out_specs=[
            pl.BlockSpec((rows_blk, D), lambda i: (i, 0)),
            pl.BlockSpec((rows_blk // 128, 128), lambda i: (i, 0)),
        ],
        out_shape=[
            jax.ShapeDtypeStruct((rows_out, D), F32),
            jax.ShapeDtypeStruct((rows_out // 128, 128), F32),
        ],
    )(x, w, a8)


def _sc_body(xdst_hbm, ar_hbm, al_hbm, seg2d_hbm, zerov_hbm, zeron_hbm,
             outv_hbm, outn_hbm, al_v, xd_v, idx_v, ar_v, w_v, wp_v,
             accv_sh, accn_sh):
    c = lax.axis_index("c")
    s = lax.axis_index("s")
    wid = s * 2 + c
    rows_per_tile = NP // 16
    rtile = pl.ds(s * rows_per_tile, rows_per_tile)
    iota = lax.iota(I32, 16)
    zf16 = jnp.zeros((16,), F32)
    lane0 = (iota == 0).astype(F32)

    pltpu.sync_copy(zerov_hbm.at[rtile], accv_sh.at[rtile])
    pltpu.sync_copy(zeron_hbm.at[rtile], accn_sh.at[rtile])
    pltpu.sync_copy(al_hbm, al_v)
    plsc.subcore_barrier()

    base0 = wid * CHUNK

    def _block(blk, carry):
        base = base0 + blk * BLK
        pltpu.sync_copy(xdst_hbm.at[pl.ds(base, BLK)], xd_v)
        pltpu.sync_copy(ar_hbm.at[pl.ds(base, BLK)], ar_v)
        pltpu.sync_copy(seg2d_hbm.at[wid * NBLK + blk], idx_v)

        for k in range(BLK // 16):
            sl = pl.ds(k * 16, 16)
            seg16 = idx_v[0, sl]
            alv = plsc.load_gather(al_v, [seg16])
            z = alv + ar_v[sl]
            w_v[sl] = jnp.exp(jnp.maximum(z, SLOPE * z))

        def _egroup(g, carry2):
            wvec = w_v[pl.ds(g * 16, 16)]
            for t in range(16):
                e = g * 16 + t
                vw = zf16 + wvec[t]
                for k8 in range(8):
                    sl = pl.ds(k8 * 16, 16)
                    xd_v[e, sl] = xd_v[e, sl] * vw
                wp_v[e, pl.ds(0, 16)] = vw * lane0
            return carry2
        lax.fori_loop(0, BLK // 16, _egroup, 0)

        pltpu.sync_copy(xd_v, accv_sh.at[idx_v.at[0]], add=True)
        pltpu.sync_copy(wp_v, accn_sh.at[idx_v.at[0]], add=True)
        return carry

    lax.fori_loop(0, NBLK, _block, 0)

    plsc.subcore_barrier()
    pltpu.sync_copy(accv_sh.at[rtile], outv_hbm.at[c, rtile])
    pltpu.sync_copy(accn_sh.at[rtile], outn_hbm.at[c, rtile])


def _sc_aggregate(x_dst, ar_flat, al_flat, seg2d):
    mesh = plsc.VectorSubcoreMesh(core_axis_name="c", subcore_axis_name="s")
    zerov = jnp.zeros((NP, D), F32)
    zeron = jnp.zeros((NP, 16), F32)
    kern = functools.partial(
        pl.kernel,
        out_type=(jax.ShapeDtypeStruct((2, NP, D), F32),
                  jax.ShapeDtypeStruct((2, NP, 16), F32)),
        mesh=mesh,
        compiler_params=pltpu.CompilerParams(use_tc_tiling_on_sc=False,
                                             needs_layout_passes=False),
        scratch_types=[
            pltpu.VMEM((NP,), F32),
            pltpu.VMEM((BLK, D), F32),
            pltpu.VMEM((1, 128), I32),
            pltpu.VMEM((BLK,), F32),
            pltpu.VMEM((BLK,), F32),
            pltpu.VMEM((BLK, 16), F32),
            pltpu.VMEM_SHARED((NP, D), F32),
            pltpu.VMEM_SHARED((NP, 16), F32),
        ],
    )(_sc_body)
    return kern(x_dst, ar_flat, al_flat, seg2d, zerov, zeron)


def _final_body(accv_ref, accn_ref, xs_ref, sa_ref, da_ref, o_ref):
    vec = accv_ref[0] + accv_ref[1]
    an = accn_ref[0] + accn_ref[1]
    norm = an[:, 0:1]
    xs = xs_ref[...]
    wsum = sa_ref[0:1, :] + da_ref[0:1, :]
    s2 = jnp.sum(xs * wsum, axis=1, keepdims=True)
    self_un = jnp.exp(jnp.maximum(s2, SLOPE * s2))
    o_ref[...] = (vec + self_un * xs) / (norm + self_un)


def _tc_final(accv, accn, x_src, sa, da):
    rows_blk = 1024
    grid = NP // rows_blk
    sa8 = jnp.broadcast_to(sa.reshape(1, D), (8, D))
    da8 = jnp.broadcast_to(da.reshape(1, D), (8, D))
    return pl.pallas_call(
        _final_body,
        grid=(grid,),
        in_specs=[
            pl.BlockSpec((2, rows_blk, D), lambda i: (0, i, 0)),
            pl.BlockSpec((2, rows_blk, 16), lambda i: (0, i, 0)),
            pl.BlockSpec((rows_blk, D), lambda i: (i, 0)),
            pl.BlockSpec((8, D), lambda i: (0, 0)),
            pl.BlockSpec((8, D), lambda i: (0, 0)),
        ],
        out_specs=pl.BlockSpec((rows_blk, D), lambda i: (i, 0)),
        out_shape=jax.ShapeDtypeStruct((NP, D), F32),
    )(accv, accn, x_src, sa8, da8)


def kernel(src, edge, dst, segment_ids, W, src_attn, dst_attn):
    del edge
    sa = src_attn.reshape(D)
    da = dst_attn.reshape(D)

    src_p = jnp.concatenate([src, jnp.zeros((NP - N, D), F32)], axis=0)
    seg_p = jnp.concatenate(
        [segment_ids.astype(I32), jnp.full((EP - E,), N, I32)], axis=0)
    seg2d = seg_p.reshape(NW * NBLK, 1, 128)

    x_dst, ar2d = _tc_project(dst, W, da, 4096, EP)
    x_src, al2d = _tc_project(src_p, W, sa, 2048, NP)

    accv, accn = _sc_aggregate(x_dst, ar2d.reshape(EP), al2d.reshape(NP),
                               seg2d)
    out = _tc_final(accv, accn, x_src, sa, da)
    return out[:N]


# speedup vs baseline: 11.6839x; 1.4255x over previous
import functools

import jax
import jax.numpy as jnp
from jax import lax
from jax.experimental import pallas as pl
from jax.experimental.pallas import tpu as pltpu
from jax.experimental.pallas import tpu_sc as plsc

N = 10000
E = 320000
D = 128
C = 128
SLOPE = 0.2

NP = 10240
EP = 327680
ACC_COLS = 144

NW = 32
CHUNK = EP // NW
BLK = 128
NBLK = CHUNK // BLK

F32 = jnp.float32
I32 = jnp.int32


def _mm_body(x_ref, w_ref, a_ref, xo_ref, ao_ref):
    x = x_ref[...]
    xw = jnp.dot(x, w_ref[...], preferred_element_type=F32)
    xo_ref[...] = xw
    a_row = a_ref[0:1, :]
    a1d = jnp.sum(xw * a_row, axis=1)
    ao_ref[...] = a1d.reshape(ao_ref.shape)


def _tc_project(x, w, attn_vec, rows_blk, rows_out):
    rows = x.shape[0]
    grid = (rows + rows_blk - 1) // rows_blk
    a8 = jnp.broadcast_to(attn_vec.reshape(1, D), (8, D))
    return pl.pallas_call(
        _mm_body,
        grid=(grid,),
        in_specs=[
            pl.BlockSpec((rows_blk, D), lambda i: (i, 0)),
            pl.BlockSpec((D, D), lambda i: (0, 0)),
            pl.BlockSpec((8, D), lambda i: (0, 0)),
        ],
        out_specs=[
            pl.BlockSpec((rows_blk, D), lambda i: (i, 0)),
            pl.BlockSpec((rows_blk // 128, 128), lambda i: (i, 0)),
        ],
        out_shape=[
            jax.ShapeDtypeStruct((rows_out, D), F32),
            jax.ShapeDtypeStruct((rows_out // 128, 128), F32),
        ],
    )(x, w, a8)


NH = NP // 2
ACC_R = NH + 16
NBUF = 3


def _sc_body(xdst_hbm, ar_hbm, al_hbm, seg_hbm, cut_hbm, zerov_hbm,
             zeron_hbm, outv_hbm, outn_hbm,
             al_v, cut_v, w_v, xd_v, seg_v, lidx_v, ar_v, wp_v,
             lsem, vsem, nsem, accv_sh, accn_sh):
    c = lax.axis_index("c")
    s = lax.axis_index("s")
    rpt = NH // 16
    rtile = pl.ds(s * rpt, rpt)
    iota = lax.iota(I32, 16)
    zf16 = jnp.zeros((16,), F32)
    lane0 = (iota == 0).astype(F32)

    pltpu.sync_copy(zerov_hbm.at[rtile], accv_sh.at[rtile])
    pltpu.sync_copy(zeron_hbm.at[rtile], accn_sh.at[rtile])
    pltpu.sync_copy(al_hbm, al_v)
    pltpu.sync_copy(cut_hbm, cut_v)
    plsc.subcore_barrier()

    cut = cut_v[...][0]
    start = jnp.where(c == 0, 0, cut)
    end = jnp.where(c == 0, cut, EP)
    lenc = end - start
    my_lo = start + (lenc * s) // 16
    my_hi = start + (lenc * (s + 1)) // 16
    alo = (my_lo // 8) * 8
    nblk = (my_hi - alo + BLK - 1) // BLK
    off = c * NH

    def _ab(blk):
        return jnp.minimum(alo + blk * BLK, EP - BLK)

    def _start_load(blk, b):
        ab = _ab(blk)
        pltpu.async_copy(xdst_hbm.at[pl.ds(ab, BLK)], xd_v[b], lsem[b])
        pltpu.async_copy(ar_hbm.at[pl.ds(ab, BLK)], ar_v[b], lsem[b])
        pltpu.async_copy(seg_hbm.at[pl.ds(ab, BLK)], seg_v[b], lsem[b])

    def _wait_load(b):
        pltpu.make_async_copy(xdst_hbm.at[pl.ds(0, BLK)], xd_v[b], lsem[b]).wait()
        pltpu.make_async_copy(ar_hbm.at[pl.ds(0, BLK)], ar_v[b], lsem[b]).wait()
        pltpu.make_async_copy(seg_hbm.at[pl.ds(0, BLK)], seg_v[b], lsem[b]).wait()

    def _wait_stream(b):
        pltpu.make_async_copy(xdst_hbm.at[pl.ds(0, BLK)], xd_v[b], vsem[b]).wait()
        pltpu.make_async_copy(zeron_hbm.at[pl.ds(0, BLK)], wp_v[b], nsem[b]).wait()

    for b in range(2):
        @pl.when(b < nblk)
        def _():
            _start_load(b, b)

    def _outer(g, carry):
        for b in range(NBUF):
            blk = g * NBUF + b

            @pl.when(blk < nblk)
            def _():
                abu = alo + blk * BLK
                ab = _ab(blk)
                lmax = jnp.maximum(my_lo, abu)
                _wait_load(b)

                for k in range(BLK // 16):
                    sl = pl.ds(k * 16, 16)
                    ei = ab + k * 16 + iota
                    seg16 = seg_v[b][sl]
                    valid = (ei >= lmax) & (ei < my_hi)
                    alv = plsc.load_gather(al_v, [seg16])
                    z = alv + ar_v[b][sl]
                    w = jnp.exp(jnp.maximum(z, SLOPE * z))
                    w_v[sl] = jnp.where(valid, w, 0.0)
                    lidx_v[b][sl] = jnp.where(valid, seg16 - off, NH)

                def _egroup(gg, carry2):
                    wvec = w_v[pl.ds(gg * 16, 16)]
                    for t in range(16):
                        e = gg * 16 + t
                        vw = zf16 + wvec[t]
                        for k8 in range(8):
                            sl = pl.ds(k8 * 16, 16)
                            xd_v[b][e, sl] = xd_v[b][e, sl] * vw
                        wp_v[b][e, pl.ds(0, 16)] = vw * lane0
                    return carry2
                lax.fori_loop(0, BLK // 16, _egroup, 0)

                nb = (b + 2) % NBUF

                @pl.when(blk + 2 < nblk)
                def _():
                    @pl.when(blk >= 1)
                    def _():
                        _wait_stream(nb)
                    _start_load(blk + 2, nb)

                pltpu.async_copy(xd_v[b], accv_sh.at[lidx_v[b]], vsem[b],
                                 add=True)
                pltpu.async_copy(wp_v[b], accn_sh.at[lidx_v[b]], nsem[b],
                                 add=True)
        return carry

    lax.fori_loop(0, (nblk + NBUF - 1) // NBUF, _outer, 0)

    for b in range(NBUF):
        @pl.when(b < nblk)
        def _():
            _wait_stream(b)

    plsc.subcore_barrier()
    pltpu.sync_copy(accv_sh.at[rtile], outv_hbm.at[c, rtile])
    pltpu.sync_copy(accn_sh.at[rtile], outn_hbm.at[c, rtile])


def _sc_aggregate(x_dst, ar_flat, al_flat, seg_p, cut16):
    mesh = plsc.VectorSubcoreMesh(core_axis_name="c", subcore_axis_name="s")
    zerov = jnp.zeros((NH, D), F32)
    zeron = jnp.zeros((NH, 16), F32)
    kern = functools.partial(
        pl.kernel,
        out_type=(jax.ShapeDtypeStruct((2, NH, D), F32),
                  jax.ShapeDtypeStruct((2, NH, 16), F32)),
        mesh=mesh,
        compiler_params=pltpu.CompilerParams(use_tc_tiling_on_sc=False,
                                             needs_layout_passes=False),
        scratch_types=[
            pltpu.VMEM((NP,), F32),
            pltpu.VMEM((16,), I32),
            pltpu.VMEM((BLK,), F32),
            [pltpu.VMEM((BLK, D), F32)] * NBUF,
            [pltpu.VMEM((BLK,), I32)] * NBUF,
            [pltpu.VMEM((BLK,), I32)] * NBUF,
            [pltpu.VMEM((BLK,), F32)] * NBUF,
            [pltpu.VMEM((BLK, 16), F32)] * NBUF,
            [pltpu.SemaphoreType.DMA] * NBUF,
            [pltpu.SemaphoreType.DMA] * NBUF,
            [pltpu.SemaphoreType.DMA] * NBUF,
            pltpu.VMEM_SHARED((ACC_R, D), F32),
            pltpu.VMEM_SHARED((ACC_R, 16), F32),
        ],
    )(_sc_body)
    return kern(x_dst, ar_flat, al_flat, seg_p, cut16, zerov, zeron)


def _final_body(accv_ref, accn_ref, xs_ref, sa_ref, da_ref, o_ref):
    vec = accv_ref[...]
    norm = accn_ref[:, 0:1]
    xs = xs_ref[...]
    wsum = sa_ref[0:1, :] + da_ref[0:1, :]
    s2 = jnp.sum(xs * wsum, axis=1, keepdims=True)
    self_un = jnp.exp(jnp.maximum(s2, SLOPE * s2))
    o_ref[...] = (vec + self_un * xs) / (norm + self_un)


def _tc_final(accv, accn, x_src, sa, da):
    rows_blk = 1024
    grid = NP // rows_blk
    sa8 = jnp.broadcast_to(sa.reshape(1, D), (8, D))
    da8 = jnp.broadcast_to(da.reshape(1, D), (8, D))
    return pl.pallas_call(
        _final_body,
        grid=(grid,),
        in_specs=[
            pl.BlockSpec((rows_blk, D), lambda i: (i, 0)),
            pl.BlockSpec((rows_blk, 16), lambda i: (i, 0)),
            pl.BlockSpec((rows_blk, D), lambda i: (i, 0)),
            pl.BlockSpec((8, D), lambda i: (0, 0)),
            pl.BlockSpec((8, D), lambda i: (0, 0)),
        ],
        out_specs=pl.BlockSpec((rows_blk, D), lambda i: (i, 0)),
        out_shape=jax.ShapeDtypeStruct((NP, D), F32),
    )(accv, accn, x_src, sa8, da8)


def kernel(src, edge, dst, segment_ids, W, src_attn, dst_attn):
    del edge
    sa = src_attn.reshape(D)
    da = dst_attn.reshape(D)

    src_p = jnp.concatenate([src, jnp.zeros((NP - N, D), F32)], axis=0)
    seg_p = jnp.concatenate(
        [segment_ids.astype(I32), jnp.full((EP - E,), N, I32)], axis=0)
    cut = jnp.searchsorted(segment_ids, NH).astype(I32)
    cut16 = jnp.full((16,), 1, I32) * cut

    x_dst, ar2d = _tc_project(dst, W, da, 4096, EP)
    x_src, al2d = _tc_project(src_p, W, sa, 2048, NP)

    accv, accn = _sc_aggregate(x_dst, ar2d.reshape(EP), al2d.reshape(NP),
                               seg_p, cut16)
    out = _tc_final(accv.reshape(NP, D), accn.reshape(NP, 16), x_src, sa, da)
    return out[:N]
